# Initial kernel scaffold; baseline (speedup 1.0000x reference)
import jax
import jax.numpy as jnp
from jax.experimental import pallas as pl


def kernel(xyz, params):
    raise NotImplementedError("write your pallas kernel here")


# speedup vs baseline: 13.1907x; 13.1907x over previous
import functools

import jax
import jax.numpy as jnp
from jax import lax
from jax.experimental import pallas as pl
from jax.experimental.pallas import tpu as pltpu
from jax.experimental.pallas import tpu_sc as plsc

_INTERPRET = False

_B = 16
_N1 = 4096
_S1 = 256
_S2 = 64
_K = 16
_R1SQ = 0.025 ** 2
_R2SQ = 0.05 ** 2


def _fps_body(npoint, n, xs_ref, ys_ref, zs_ref, qx_ref, qy_ref, qz_ref):
    b = xs_ref.shape[0]
    xs = xs_ref[...]
    ys = ys_ref[...]
    zs = zs_ref[...]
    iota = lax.broadcasted_iota(jnp.int32, (1, n), 1)
    iota_np = lax.broadcasted_iota(jnp.int32, (1, npoint), 1)

    def body(i, state):
        distance, far, qx, qy, qz = state
        eq = iota == far
        cx = jnp.sum(jnp.where(eq, xs, 0.0), axis=1, keepdims=True)
        cy = jnp.sum(jnp.where(eq, ys, 0.0), axis=1, keepdims=True)
        cz = jnp.sum(jnp.where(eq, zs, 0.0), axis=1, keepdims=True)
        sel = iota_np == i
        qx = jnp.where(sel, cx, qx)
        qy = jnp.where(sel, cy, qy)
        qz = jnp.where(sel, cz, qz)
        dx = xs - cx
        dy = ys - cy
        dz = zs - cz
        d = dx * dx + dy * dy
        d = d + dz * dz
        distance = jnp.where(d < distance, d, distance)
        m = jnp.max(distance, axis=1, keepdims=True)
        far = jnp.min(jnp.where(distance == m, iota, n), axis=1, keepdims=True)
        return distance, far, qx, qy, qz

    init = (jnp.full((b, n), 1e10, jnp.float32),
            jnp.zeros((b, 1), jnp.int32),
            jnp.zeros((b, npoint), jnp.float32),
            jnp.zeros((b, npoint), jnp.float32),
            jnp.zeros((b, npoint), jnp.float32))
    _, _, qx, qy, qz = lax.fori_loop(0, npoint, body, init)
    qx_ref[...] = qx
    qy_ref[...] = qy
    qz_ref[...] = qz


def _fps(xs, ys, zs, npoint):
    b, n = xs.shape
    out = jax.ShapeDtypeStruct((b, npoint), jnp.float32)
    return pl.pallas_call(
        functools.partial(_fps_body, npoint, n),
        out_shape=(out, out, out),
        interpret=_INTERPRET,
    )(xs, ys, zs)


def _select_body(nsample, r2, n, xs_ref, ys_ref, zs_ref,
                 qx_ref, qy_ref, qz_ref, idx_ref):
    bprog = pl.program_id(0)
    xs = xs_ref[0]
    ys = ys_ref[0]
    zs = zs_ref[0]
    qx = qx_ref[0]
    qy = qy_ref[0]
    qz = qz_ref[0]
    iota = lax.broadcasted_iota(jnp.int32, (1, n), 1)
    t = qx * xs
    t = t + qy * ys
    t = t + qz * zs
    qq = qx * qx + qy * qy + qz * qz
    pp = xs * xs + ys * ys + zs * zs
    d = (-2.0 * t + qq) + pp
    cand = jnp.where(d <= r2, jnp.broadcast_to(iota, d.shape), n)
    cols = []
    for _ in range(nsample):
        m = jnp.min(cand, axis=1, keepdims=True)
        cols.append(m)
        cand = jnp.where(cand == m, n, cand)
    idx = jnp.concatenate(cols, axis=1)
    first = idx[:, 0:1]
    idx = jnp.where(idx == n, first, idx)
    idx_ref[0] = idx + bprog * n


def _select(xs, ys, zs, qx, qy, qz, r2, nsample):
    b, n = xs.shape
    s = qx.shape[1]
    pt_spec = pl.BlockSpec((1, 1, n), lambda i: (i, 0, 0))
    q_spec = pl.BlockSpec((1, s, 1), lambda i: (i, 0, 0))
    return pl.pallas_call(
        functools.partial(_select_body, nsample, r2, n),
        grid=(b,),
        in_specs=[pt_spec, pt_spec, pt_spec, q_spec, q_spec, q_spec],
        out_specs=pl.BlockSpec((1, s, nsample), lambda i: (i, 0, 0)),
        out_shape=jax.ShapeDtypeStruct((b, s, nsample), jnp.int32),
        compiler_params=pltpu.CompilerParams(
            dimension_semantics=("parallel",)),
        interpret=_INTERPRET,
    )(xs[:, None, :], ys[:, None, :], zs[:, None, :],
      qx[:, :, None], qy[:, :, None], qz[:, :, None])


def _sc_gather(table, idx):
    m = idx.shape[0]
    d = table.shape[1]
    info = plsc.get_sparse_core_info()
    nw = info.num_cores * info.num_subcores
    bpw = m // nw
    nchunk = bpw // 128
    idx2d = idx.reshape(m // 128, 128)
    mesh = plsc.VectorSubcoreMesh(core_axis_name="c", subcore_axis_name="s")

    @functools.partial(
        pl.kernel, mesh=mesh,
        out_type=jax.ShapeDtypeStruct((m, d), jnp.float32),
        scratch_types=[
            pltpu.VMEM((nchunk, 128), jnp.int32),
            pltpu.VMEM((bpw, d), jnp.float32),
            pltpu.SemaphoreType.DMA,
        ],
        compiler_params=pltpu.CompilerParams(use_tc_tiling_on_sc=False),
    )
    def gather_kernel(table_hbm, idx_hbm, out_hbm, idx_v, rows_v, sem):
        wid = lax.axis_index("s") * info.num_cores + lax.axis_index("c")
        pltpu.sync_copy(idx_hbm.at[pl.ds(wid * nchunk, nchunk)], idx_v)
        copies = [
            pltpu.async_copy(table_hbm.at[idx_v.at[j]],
                             rows_v.at[pl.ds(j * 128, 128)], sem)
            for j in range(nchunk)
        ]
        for c in copies:
            c.wait()
        pltpu.sync_copy(rows_v, out_hbm.at[pl.ds(wid * bpw, bpw)])

    return gather_kernel(table, idx2d)


def _sa_body(reps, g_ref, q_ref, w1_ref, b1_ref, w2_ref, b2_ref,
             w3_ref, b3_ref, out_ref):
    g = g_ref[0]
    q = q_ref[0]
    s, dp = q.shape
    rep = jnp.broadcast_to(q[:, None, :], (s, reps, dp)).reshape(s * reps, dp)
    x = g - rep
    h = jnp.maximum(
        jnp.dot(x, w1_ref[...], preferred_element_type=jnp.float32)
        + b1_ref[...], 0.0)
    h = jnp.maximum(
        jnp.dot(h, w2_ref[...], preferred_element_type=jnp.float32)
        + b2_ref[...], 0.0)
    h = jnp.maximum(
        jnp.dot(h, w3_ref[...], preferred_element_type=jnp.float32)
        + b3_ref[...], 0.0)
    c = h.shape[-1]
    pooled = jnp.max(h.reshape(s, reps, c), axis=1)
    out_ref[0] = jnp.concatenate([q[:, :16], pooled], axis=-1)


def _sa_mlp(g, q, w1, b1, w2, b2, w3, b3, reps):
    b, sk, dp = g.shape
    s = sk // reps
    cout = w3.shape[1]

    def wspec(w):
        nd = w.ndim
        return pl.BlockSpec(w.shape, lambda i: (0,) * nd)

    return pl.pallas_call(
        functools.partial(_sa_body, reps),
        grid=(b,),
        in_specs=[
            pl.BlockSpec((1, sk, dp), lambda i: (i, 0, 0)),
            pl.BlockSpec((1, s, dp), lambda i: (i, 0, 0)),
            wspec(w1), wspec(b1), wspec(w2), wspec(b2),
            wspec(w3), wspec(b3),
        ],
        out_specs=pl.BlockSpec((1, s, 16 + cout), lambda i: (i, 0, 0)),
        out_shape=jax.ShapeDtypeStruct((b, s, 16 + cout), jnp.float32),
        compiler_params=pltpu.CompilerParams(
            dimension_semantics=("parallel",)),
        interpret=_INTERPRET,
    )(g, q, w1, b1, w2, b2, w3, b3)


def _tail_body(t2_ref, t3_ref, wc3_ref, b31_ref, w32_ref, b32_ref,
               w33_ref, b33_ref, wf1x_ref, wf1b_ref, bf1_ref,
               wf2_ref, bf2_ref, wh1_ref, bh1_ref, wh2_ref, bh2_ref,
               out_ref):
    t3 = t3_ref[0]
    h = jnp.maximum(
        jnp.dot(t3, wc3_ref[...], preferred_element_type=jnp.float32)
        + b31_ref[...], 0.0)
    h = jnp.maximum(
        jnp.dot(h, w32_ref[...], preferred_element_type=jnp.float32)
        + b32_ref[...], 0.0)
    h = jnp.maximum(
        jnp.dot(h, w33_ref[...], preferred_element_type=jnp.float32)
        + b33_ref[...], 0.0)
    l3 = jnp.max(h, axis=0, keepdims=True)

    t2 = t2_ref[0]
    g = jnp.maximum(
        jnp.dot(t2, wf1x_ref[...], preferred_element_type=jnp.float32)
        + jnp.dot(l3, wf1b_ref[...], preferred_element_type=jnp.float32)
        + bf1_ref[...], 0.0)
    g = jnp.maximum(
        jnp.dot(g, wf2_ref[...], preferred_element_type=jnp.float32)
        + bf2_ref[...], 0.0)
    a = jnp.maximum(
        jnp.dot(g, wh1_ref[...], preferred_element_type=jnp.float32)
        + bh1_ref[...], 0.0)
    logits = lax.dot_general(
        wh2_ref[...], a, (((1,), (1,)), ((), ())),
        preferred_element_type=jnp.float32) + bh2_ref[...]
    m = jnp.max(logits)
    e = jnp.exp(logits - m)
    out_ref[0] = e / jnp.sum(e)


def _tail(t2, t3, weights):
    b = t2.shape[0]

    def wspec(w):
        nd = w.ndim
        return pl.BlockSpec(w.shape, lambda i: (0,) * nd)

    return pl.pallas_call(
        _tail_body,
        grid=(b,),
        in_specs=[
            pl.BlockSpec((1, _S1, 144), lambda i: (i, 0, 0)),
            pl.BlockSpec((1, _S2, 272), lambda i: (i, 0, 0)),
        ] + [wspec(w) for w in weights],
        out_specs=pl.BlockSpec((1, 13, _S1), lambda i: (i, 0, 0)),
        out_shape=jax.ShapeDtypeStruct((b, 13, _S1), jnp.float32),
        compiler_params=pltpu.CompilerParams(
            dimension_semantics=("parallel",)),
        interpret=_INTERPRET,
    )(t2, t3, *weights)


def kernel(xyz, params):
    xyz = xyz[:, :, :3].astype(jnp.float32)
    b = xyz.shape[0]
    xs, ys, zs = xyz[:, :, 0], xyz[:, :, 1], xyz[:, :, 2]

    (w11, b11), (w12, b12), (w13, b13) = params['sa1']
    w11p = jnp.zeros((16, 64), jnp.float32).at[0:3].set(w11.T)
    (w21, b21), (w22, b22), (w23, b23) = params['sa2']
    w21p = jnp.zeros((144, 128), jnp.float32)
    w21p = w21p.at[0:3].set(w21[:, :3].T).at[16:144].set(w21[:, 3:].T)
    (w31, b31), (w32, b32), (w33, b33) = params['sa3']
    w31p = jnp.zeros((272, 256), jnp.float32)
    w31p = w31p.at[0:3].set(w31[:, :3].T).at[16:272].set(w31[:, 3:].T)
    (wf1, bf1), (wf2, bf2) = params['fp2']
    wf1x = jnp.zeros((144, 256), jnp.float32).at[16:144].set(wf1[:, :128].T)
    wf1b = wf1[:, 128:].T
    (wh1, bh1), (wh2, bh2) = params['head']

    def row(v):
        return v[None, :]

    nx, ny, nz = _fps(xs, ys, zs, _S1)
    idx1 = _select(xs, ys, zs, nx, ny, nz, _R1SQ, _K)
    table1 = jnp.pad(xyz, ((0, 0), (0, 0), (0, 13))).reshape(b * _N1, 16)
    g1 = _sc_gather(table1, idx1.reshape(-1))
    nx16 = jnp.pad(jnp.stack([nx, ny, nz], axis=-1),
                   ((0, 0), (0, 0), (0, 13)))
    t2 = _sa_mlp(g1.reshape(b, _S1 * _K, 16), nx16,
                 w11p, row(b11), w12.T, row(b12), w13.T, row(b13), _K)

    mx, my, mz = _fps(nx, ny, nz, _S2)
    idx2 = _select(nx, ny, nz, mx, my, mz, _R2SQ, _K)
    g2 = _sc_gather(t2.reshape(b * _S1, 144), idx2.reshape(-1))
    mx144 = jnp.pad(jnp.stack([mx, my, mz], axis=-1),
                    ((0, 0), (0, 0), (0, 141)))
    t3 = _sa_mlp(g2.reshape(b, _S2 * _K, 144), mx144,
                 w21p, row(b21), w22.T, row(b22), w23.T, row(b23), _K)

    weights = [w31p, row(b31), w32.T, row(b32), w33.T, row(b33),
               wf1x, wf1b, row(bf1), wf2.T, row(bf2),
               wh1.T, row(bh1), wh2, bh2[:, None]]
    out = _tail(t2, t3, weights)
    return out.reshape(b, 13 * _S1)


# speedup vs baseline: 13.4801x; 1.0219x over previous
import functools

import jax
import jax.numpy as jnp
from jax import lax
from jax.experimental import pallas as pl
from jax.experimental.pallas import tpu as pltpu
from jax.experimental.pallas import tpu_sc as plsc

_INTERPRET = False

_B = 16
_N1 = 4096
_S1 = 256
_S2 = 64
_K = 16
_R1SQ = 0.025 ** 2
_R2SQ = 0.05 ** 2


def _fps_body(npoint, n, xs_ref, ys_ref, zs_ref, qx_ref, qy_ref, qz_ref):
    b = xs_ref.shape[0]
    xs = xs_ref[...]
    ys = ys_ref[...]
    zs = zs_ref[...]
    b = xs.shape[0]
    xyz3 = jnp.concatenate([xs, ys, zs], axis=0)
    iota = lax.broadcasted_iota(jnp.int32, (1, n), 1)
    iota_np = lax.broadcasted_iota(jnp.int32, (1, npoint), 1)

    def body(i, state):
        distance, far, qx, qy, qz = state
        eq = iota == far
        eq3 = jnp.concatenate([eq, eq, eq], axis=0)
        cxyz = jnp.sum(jnp.where(eq3, xyz3, 0.0), axis=1, keepdims=True)
        cx = cxyz[0:b]
        cy = cxyz[b:2 * b]
        cz = cxyz[2 * b:3 * b]
        sel = iota_np == i
        qx = jnp.where(sel, cx, qx)
        qy = jnp.where(sel, cy, qy)
        qz = jnp.where(sel, cz, qz)
        dx = xs - cx
        dy = ys - cy
        dz = zs - cz
        d = dx * dx + dy * dy
        d = d + dz * dz
        distance = jnp.minimum(distance, d)
        m = jnp.max(distance, axis=1, keepdims=True)
        far = jnp.min(jnp.where(distance == m, iota, n), axis=1, keepdims=True)
        return distance, far, qx, qy, qz

    init = (jnp.full((b, n), 1e10, jnp.float32),
            jnp.zeros((b, 1), jnp.int32),
            jnp.zeros((b, npoint), jnp.float32),
            jnp.zeros((b, npoint), jnp.float32),
            jnp.zeros((b, npoint), jnp.float32))
    _, _, qx, qy, qz = lax.fori_loop(0, npoint, body, init)
    qx_ref[...] = qx
    qy_ref[...] = qy
    qz_ref[...] = qz


def _fps(xs, ys, zs, npoint):
    b, n = xs.shape
    out = jax.ShapeDtypeStruct((b, npoint), jnp.float32)
    return pl.pallas_call(
        functools.partial(_fps_body, npoint, n),
        out_shape=(out, out, out),
        interpret=_INTERPRET,
    )(xs, ys, zs)


def _select_body(bp, nsample, r2, n, xs_ref, ys_ref, zs_ref,
                 qx_ref, qy_ref, qz_ref, idx_ref):
    bprog = pl.program_id(0)
    s = qx_ref.shape[1]
    iota = lax.broadcasted_iota(jnp.int32, (1, 1, n), 2)
    ones_col = jnp.ones((s, 1), jnp.float32)
    zeros_col = jnp.zeros((s, 1), jnp.float32)
    zeros_row = jnp.zeros((3, n), jnp.float32)
    cands = []
    for bi in range(bp):
        xs = xs_ref[bi]
        ys = ys_ref[bi]
        zs = zs_ref[bi]
        qx = qx_ref[bi]
        qy = qy_ref[bi]
        qz = qz_ref[bi]
        qq = qx * qx + qy * qy + qz * qz
        pp = xs * xs + ys * ys + zs * zs
        a_mat = jnp.concatenate(
            [qx, qy, qz, qq, ones_col, zeros_col, zeros_col, zeros_col],
            axis=1)
        b_mat = jnp.concatenate(
            [-2.0 * xs, -2.0 * ys, -2.0 * zs, jnp.ones((1, n), jnp.float32),
             pp, zeros_row], axis=0)
        d = jnp.dot(a_mat, b_mat, preferred_element_type=jnp.float32)
        cands.append(jnp.where(d <= r2, jnp.broadcast_to(iota[0], d.shape), n))
    cand = jnp.stack(cands, axis=0)
    cols = []
    for _ in range(nsample):
        m = jnp.min(cand, axis=2, keepdims=True)
        cols.append(m)
        cand = jnp.where(cand == m, n, cand)
    idx = jnp.concatenate(cols, axis=2)
    first = idx[:, :, 0:1]
    idx = jnp.where(idx == n, first, idx)
    offs = (bprog * bp + lax.broadcasted_iota(jnp.int32, (bp, 1, 1), 0)) * n
    idx_ref[...] = idx + offs


def _select(xs, ys, zs, qx, qy, qz, r2, nsample, bp):
    b, n = xs.shape
    s = qx.shape[1]
    pt_spec = pl.BlockSpec((bp, 1, n), lambda i: (i, 0, 0))
    q_spec = pl.BlockSpec((bp, s, 1), lambda i: (i, 0, 0))
    return pl.pallas_call(
        functools.partial(_select_body, bp, nsample, r2, n),
        grid=(b // bp,),
        in_specs=[pt_spec, pt_spec, pt_spec, q_spec, q_spec, q_spec],
        out_specs=pl.BlockSpec((bp, s, nsample), lambda i: (i, 0, 0)),
        out_shape=jax.ShapeDtypeStruct((b, s, nsample), jnp.int32),
        compiler_params=pltpu.CompilerParams(
            dimension_semantics=("parallel",),
            vmem_limit_bytes=100 * 1024 * 1024),
        interpret=_INTERPRET,
    )(xs[:, None, :], ys[:, None, :], zs[:, None, :],
      qx[:, :, None], qy[:, :, None], qz[:, :, None])


def _sc_gather(table, idx):
    m = idx.shape[0]
    d = table.shape[1]
    info = plsc.get_sparse_core_info()
    nw = info.num_cores * info.num_subcores
    bpw = m // nw
    nchunk = bpw // 128
    idx2d = idx.reshape(m // 128, 128)
    mesh = plsc.VectorSubcoreMesh(core_axis_name="c", subcore_axis_name="s")

    @functools.partial(
        pl.kernel, mesh=mesh,
        out_type=jax.ShapeDtypeStruct((m, d), jnp.float32),
        scratch_types=[
            pltpu.VMEM((nchunk, 128), jnp.int32),
            pltpu.VMEM((bpw, d), jnp.float32),
            pltpu.SemaphoreType.DMA,
        ],
        compiler_params=pltpu.CompilerParams(use_tc_tiling_on_sc=False),
    )
    def gather_kernel(table_hbm, idx_hbm, out_hbm, idx_v, rows_v, sem):
        wid = lax.axis_index("s") * info.num_cores + lax.axis_index("c")
        pltpu.sync_copy(idx_hbm.at[pl.ds(wid * nchunk, nchunk)], idx_v)
        copies = [
            pltpu.async_copy(table_hbm.at[idx_v.at[j]],
                             rows_v.at[pl.ds(j * 128, 128)], sem)
            for j in range(nchunk)
        ]
        for c in copies:
            c.wait()
        pltpu.sync_copy(rows_v, out_hbm.at[pl.ds(wid * bpw, bpw)])

    return gather_kernel(table, idx2d)


def _sa_body(reps, g_ref, q_ref, w1_ref, b1_ref, w2_ref, b2_ref,
             w3_ref, b3_ref, out_ref):
    bp, sk, dp = g_ref.shape
    g = g_ref[...].reshape(bp * sk, dp)
    q = q_ref[...].reshape(bp * q_ref.shape[1], dp)
    s = q.shape[0]
    rep = jnp.broadcast_to(q[:, None, :], (s, reps, dp)).reshape(s * reps, dp)
    x = g - rep
    h = jnp.maximum(
        jnp.dot(x, w1_ref[...], preferred_element_type=jnp.float32)
        + b1_ref[...], 0.0)
    h = jnp.maximum(
        jnp.dot(h, w2_ref[...], preferred_element_type=jnp.float32)
        + b2_ref[...], 0.0)
    h = jnp.maximum(
        jnp.dot(h, w3_ref[...], preferred_element_type=jnp.float32)
        + b3_ref[...], 0.0)
    c = h.shape[-1]
    pooled = jnp.max(h.reshape(s, reps, c), axis=1)
    res = jnp.concatenate([q[:, :16], pooled], axis=-1)
    out_ref[...] = res.reshape(bp, s // bp, 16 + c)


def _sa_mlp(g, q, w1, b1, w2, b2, w3, b3, reps, bp):
    b, sk, dp = g.shape
    s = sk // reps
    cout = w3.shape[1]

    def wspec(w):
        nd = w.ndim
        return pl.BlockSpec(w.shape, lambda i: (0,) * nd)

    return pl.pallas_call(
        functools.partial(_sa_body, reps),
        grid=(b // bp,),
        in_specs=[
            pl.BlockSpec((bp, sk, dp), lambda i: (i, 0, 0)),
            pl.BlockSpec((bp, s, dp), lambda i: (i, 0, 0)),
            wspec(w1), wspec(b1), wspec(w2), wspec(b2),
            wspec(w3), wspec(b3),
        ],
        out_specs=pl.BlockSpec((bp, s, 16 + cout), lambda i: (i, 0, 0)),
        out_shape=jax.ShapeDtypeStruct((b, s, 16 + cout), jnp.float32),
        compiler_params=pltpu.CompilerParams(
            dimension_semantics=("parallel",),
            vmem_limit_bytes=100 * 1024 * 1024),
        interpret=_INTERPRET,
    )(g, q, w1, b1, w2, b2, w3, b3)


def _tail_body(nb, t2_ref, t3_ref, wc3_ref, b31_ref, w32_ref, b32_ref,
               w33_ref, b33_ref, wf1x_ref, wf1b_ref, bf1_ref,
               wf2_ref, bf2_ref, wh1_ref, bh1_ref, wh2_ref, bh2_ref,
               out_ref):
    t3 = t3_ref[...].reshape(nb * _S2, 272)
    h = jnp.maximum(
        jnp.dot(t3, wc3_ref[...], preferred_element_type=jnp.float32)
        + b31_ref[...], 0.0)
    h = jnp.maximum(
        jnp.dot(h, w32_ref[...], preferred_element_type=jnp.float32)
        + b32_ref[...], 0.0)
    h = jnp.maximum(
        jnp.dot(h, w33_ref[...], preferred_element_type=jnp.float32)
        + b33_ref[...], 0.0)
    l3 = jnp.max(h.reshape(nb, _S2, 512), axis=1)
    l3w = jnp.dot(l3, wf1b_ref[...], preferred_element_type=jnp.float32)
    l3rep = jnp.broadcast_to(
        l3w[:, None, :], (nb, _S1, 256)).reshape(nb * _S1, 256)

    t2 = t2_ref[...].reshape(nb * _S1, 144)
    g = jnp.maximum(
        jnp.dot(t2, wf1x_ref[...], preferred_element_type=jnp.float32)
        + l3rep + bf1_ref[...], 0.0)
    g = jnp.maximum(
        jnp.dot(g, wf2_ref[...], preferred_element_type=jnp.float32)
        + bf2_ref[...], 0.0)
    a = jnp.maximum(
        jnp.dot(g, wh1_ref[...], preferred_element_type=jnp.float32)
        + bh1_ref[...], 0.0)
    logits = lax.dot_general(
        wh2_ref[...], a, (((1,), (1,)), ((), ())),
        preferred_element_type=jnp.float32) + bh2_ref[...]
    for bi in range(nb):
        sub = logits[:, bi * _S1:(bi + 1) * _S1]
        m = jnp.max(sub)
        e = jnp.exp(sub - m)
        out_ref[bi] = e / jnp.sum(e)


def _tail(t2, t3, weights):
    b = t2.shape[0]
    return pl.pallas_call(
        functools.partial(_tail_body, b),
        out_shape=jax.ShapeDtypeStruct((b, 13, _S1), jnp.float32),
        interpret=_INTERPRET,
    )(t2, t3, *weights)


def kernel(xyz, params):
    xyz = xyz[:, :, :3].astype(jnp.float32)
    b = xyz.shape[0]
    xs, ys, zs = xyz[:, :, 0], xyz[:, :, 1], xyz[:, :, 2]

    (w11, b11), (w12, b12), (w13, b13) = params['sa1']
    w11p = jnp.zeros((16, 64), jnp.float32).at[0:3].set(w11.T)
    (w21, b21), (w22, b22), (w23, b23) = params['sa2']
    w21p = jnp.zeros((144, 128), jnp.float32)
    w21p = w21p.at[0:3].set(w21[:, :3].T).at[16:144].set(w21[:, 3:].T)
    (w31, b31), (w32, b32), (w33, b33) = params['sa3']
    w31p = jnp.zeros((272, 256), jnp.float32)
    w31p = w31p.at[0:3].set(w31[:, :3].T).at[16:272].set(w31[:, 3:].T)
    (wf1, bf1), (wf2, bf2) = params['fp2']
    wf1x = jnp.zeros((144, 256), jnp.float32).at[16:144].set(wf1[:, :128].T)
    wf1b = wf1[:, 128:].T
    (wh1, bh1), (wh2, bh2) = params['head']

    def row(v):
        return v[None, :]

    nx, ny, nz = _fps(xs, ys, zs, _S1)
    idx1 = _select(xs, ys, zs, nx, ny, nz, _R1SQ, _K, 4)
    table1 = jnp.pad(xyz, ((0, 0), (0, 0), (0, 13))).reshape(b * _N1, 16)
    g1 = _sc_gather(table1, idx1.reshape(-1))
    nx16 = jnp.pad(jnp.stack([nx, ny, nz], axis=-1),
                   ((0, 0), (0, 0), (0, 13)))
    t2 = _sa_mlp(g1.reshape(b, _S1 * _K, 16), nx16,
                 w11p, row(b11), w12.T, row(b12), w13.T, row(b13), _K, 4)

    mx, my, mz = _fps(nx, ny, nz, _S2)
    idx2 = _select(nx, ny, nz, mx, my, mz, _R2SQ, _K, 16)
    g2 = _sc_gather(t2.reshape(b * _S1, 144), idx2.reshape(-1))
    mx144 = jnp.pad(jnp.stack([mx, my, mz], axis=-1),
                    ((0, 0), (0, 0), (0, 141)))
    t3 = _sa_mlp(g2.reshape(b, _S2 * _K, 144), mx144,
                 w21p, row(b21), w22.T, row(b22), w23.T, row(b23), _K, 4)

    weights = [w31p, row(b31), w32.T, row(b32), w33.T, row(b33),
               wf1x, wf1b, row(bf1), wf2.T, row(bf2),
               wh1.T, row(bh1), wh2, bh2[:, None]]
    out = _tail(t2, t3, weights)
    return out.reshape(b, 13 * _S1)
